# baseline (speedup 1.0000x reference)
import math

import jax
import jax.numpy as jnp
from jax import lax
from jax.experimental import pallas as pl
from jax.experimental.pallas import tpu as pltpu

F32 = jnp.float32
S = 2048
D = 2048
HD = 128
NH = 16
DC = 512
M_ = 4
TOPK = 256
CI = 64
NHI = 4
NG = 4
DG = 1024
RD = 64
EPS = 1e-6
SK = S // M_
RB = 256
NB = S // RB
NEG = -1e30
WBIG = 4 * HD + DC + CI + NHI + 60


PREC = lax.Precision.HIGHEST
BF16 = jnp.bfloat16


def _dot(a, b):
    return lax.dot_general(a, b, (((1,), (0,)), ((), ())),
                           precision=PREC, preferred_element_type=F32)


def _dot_t(a, b):
    return lax.dot_general(a, b, (((1,), (1,)), ((), ())),
                           precision=PREC, preferred_element_type=F32)


_DIMS = (((1,), (0,)), ((), ()))
_DIMS_T = (((1,), (1,)), ((), ()))


def _split_bf(a):
    hi = a.astype(BF16)
    lo = (a - hi.astype(F32)).astype(BF16)
    return hi, lo


def _dot1(a, b, dims=_DIMS):
    return lax.dot_general(a, b, dims, preferred_element_type=F32)


def _dot_ps(a, bh, bl, dims=_DIMS):
    ah, al = _split_bf(a)
    return (_dot1(ah, bh, dims) + _dot1(ah, bl, dims)
            + _dot1(al, bh, dims))


def _dot_h(a, b):
    bh, bl = _split_bf(b)
    return _dot_ps(a, bh, bl)


def _dot_rs(a_bf, b):
    bh, bl = _split_bf(b)
    return _dot1(a_bf, bh) + _dot1(a_bf, bl)


def _dot_bf(a, b):
    return lax.dot_general(a.astype(BF16), b.astype(BF16),
                           (((1,), (0,)), ((), ())),
                           preferred_element_type=F32)


def _dot_t_bf(a, b):
    return lax.dot_general(a.astype(BF16), b.astype(BF16),
                           (((1,), (1,)), ((), ())),
                           preferred_element_type=F32)


def _proj_kernel(x_ref, w4h_ref, w4l_ref, wdq_ref, o1_ref, o2_ref, o3_ref):
    xh, xl = _split_bf(x_ref[...])
    w4h = w4h_ref[...]
    o1_ref[...] = (_dot1(xh, w4h) + _dot1(xh, w4l_ref[...])
                   + _dot1(xl, w4h))
    p2 = _dot1(xh, wdq_ref[...])
    o2_ref[...] = p2[:, 0:DC]
    o3_ref[...] = p2[:, DC:DC + 128]


def _compress_kernel(cz_ref, xr_ref, wk_ref, ba_ref, bb_ref, g_ref, b_ref,
                     kc_ref, kp_ref):
    cz = cz_ref[...]
    c_a = cz[:, 0:HD]
    c_b = cz[:, HD:2 * HD]
    z_a = cz[:, 2 * HD:3 * HD]
    z_b = cz[:, 3 * HD:4 * HD]
    r4 = lax.broadcasted_iota(jnp.int32, (S, M_), 0)
    j4 = lax.broadcasted_iota(jnp.int32, (S, M_), 1)
    tsel = ((r4 % M_) == j4).astype(BF16)
    ba = _dot_rs(tsel, ba_ref[...])
    bb = _dot_rs(tsel, bb_ref[...])
    ea = jnp.exp(z_a + ba)
    eb = jnp.exp(z_b + bb)
    nn = lax.broadcasted_iota(jnp.int32, (SK, S), 0)
    rr = lax.broadcasted_iota(jnp.int32, (SK, S), 1)
    dm = ((rr // M_) == nn).astype(BF16)
    ds = ((rr // M_) == (nn - 1)).astype(BF16)
    num = _dot_rs(dm, ea * c_a) + _dot_rs(ds, eb * c_b)
    den = _dot_rs(dm, ea) + _dot_rs(ds, eb)
    comp = num / den
    mu = jnp.mean(comp, axis=1, keepdims=True)
    var = jnp.mean((comp - mu) ** 2, axis=1, keepdims=True)
    kc_ref[...] = ((comp - mu) * lax.rsqrt(var + EPS) * g_ref[...]
                   + b_ref[...])
    xr = xr_ref[...]
    ksum = xr[:, 0:D]
    for j in range(1, M_):
        ksum = ksum + xr[:, j * D:(j + 1) * D]
    korig = ksum * (1.0 / M_)
    kp_ref[...] = _dot_bf(korig, wk_ref[...])


def _attn_kernel(cq_ref, misc_ref, kp_ref, kc_ref, wuqh_ref, wuql_ref,
                 wiuq_ref, qng_ref, qnb_ref, sink_ref, out_ref):
    pid = pl.program_id(0)
    cq = cq_ref[...]
    qi = _dot1(cq.astype(BF16), wiuq_ref[...])
    kp = kp_ref[...].astype(BF16)
    score = jnp.zeros((RB, SK), F32)
    for h in range(NHI):
        dh = _dot1(qi[:, h * CI:(h + 1) * CI].astype(BF16), kp, _DIMS_T)
        wh = misc_ref[:, h:h + 1]
        score = score + jnp.maximum(dh, 0.0) * wh
    rows = lax.broadcasted_iota(jnp.int32, (RB, SK), 0) + pid * RB
    cols = lax.broadcasted_iota(jnp.int32, (RB, SK), 1)
    score = jnp.where(cols < rows, score, NEG)

    bits = lax.bitcast_convert_type(score, jnp.int32)
    int_min = jnp.int32(-2**31)
    code = jnp.where(bits >= 0, bits + int_min, -bits)

    def body(i, carry):
        sel, cand, krem = carry
        b = 31 - i
        bit = (lax.shift_right_logical(code, b) & 1).astype(F32)
        ones = cand * bit
        c1 = jnp.sum(ones, axis=1, keepdims=True)
        ta = (c1 < krem).astype(F32)
        sel = sel + ones * ta
        krem = krem - c1 * ta
        cand = ones + ta * (cand - 2.0 * ones)
        return sel, cand, krem

    sel0 = jnp.zeros((RB, SK), F32)
    cand0 = jnp.ones((RB, SK), F32)
    krem0 = jnp.full((RB, 1), float(TOPK), F32)
    sel, cand, krem = lax.fori_loop(0, 32, body, (sel0, cand0, krem0))
    li = lax.broadcasted_iota(jnp.int32, (SK, SK), 0)
    lj = lax.broadcasted_iota(jnp.int32, (SK, SK), 1)
    lt = (li < lj).astype(BF16)
    eqr = _dot1(cand.astype(BF16), lt)
    sel = sel + cand * (eqr < krem).astype(F32)

    keep = (sel > 0.5) & (rows < cols * M_)
    abias = jnp.where(keep, 0.0, NEG)

    cqh, cql = _split_bf(cq)
    wuqh = wuqh_ref[...]
    qall = (_dot1(cqh, wuqh) + _dot1(cqh, wuql_ref[...])
            + _dot1(cql, wuqh))
    kc = kc_ref[...]
    kch, kcl = _split_bf(kc)
    lidx = lax.broadcasted_iota(jnp.int32, (RB, HD), 1)
    srow = (lax.broadcasted_iota(jnp.int32, (RB, HD), 0) + pid * RB
            ).astype(F32)
    expo = ((lidx - RD) & (~1)).astype(F32) * (1.0 / RD)
    theta = jnp.exp(expo * (-math.log(10000.0)))
    ang = srow * theta
    cosf = jnp.where(lidx < RD, 1.0, jnp.cos(ang))
    sinf = jnp.where(lidx < RD, 0.0,
                     jnp.where((lidx & 1) == 0, -jnp.sin(ang), jnp.sin(ang)))
    even = (lidx & 1) == 0
    inv = 1.0 / math.sqrt(float(HD))
    sink = jnp.exp(sink_ref[...])
    for h in range(NH):
        qh = qall[:, h * HD:(h + 1) * HD]
        mu = jnp.mean(qh, axis=1, keepdims=True)
        var = jnp.mean((qh - mu) ** 2, axis=1, keepdims=True)
        qh = (qh - mu) * lax.rsqrt(var + EPS) * qng_ref[...] + qnb_ref[...]
        qsw = jnp.where(even, pltpu.roll(qh, HD - 1, 1),
                        pltpu.roll(qh, 1, 1))
        qh = qh * cosf + qsw * sinf
        qhh, qhl = _split_bf(qh)
        sc_h = (_dot1(qhh, kch, _DIMS_T) + _dot1(qhh, kcl, _DIMS_T)
                + _dot1(qhl, kch, _DIMS_T)) * inv + abias
        es = jnp.exp(sc_h)
        den = jnp.sum(es, axis=1, keepdims=True) + sink[:, h:h + 1]
        p = es / den
        out_ref[:, h * HD:(h + 1) * HD] = _dot_ps(p, kch, kcl)


def _out_kernel(o_ref, wodh_ref, wodl_ref, wouh_ref, woul_ref, f_ref):
    o = o_ref[...]
    acc = jnp.zeros((RB, D), F32)
    gi = (NH * HD) // NG
    for g in range(NG):
        sl = slice(g * gi, (g + 1) * gi)
        dg = _dot_ps(o[:, sl], wodh_ref[sl, :], wodl_ref[sl, :])
        su = slice(g * DG, (g + 1) * DG)
        acc = acc + _dot_ps(dg, wouh_ref[su, :], woul_ref[su, :])
    f_ref[...] = acc


def kernel(x, Wkva, Wkvb, Wza, Wzb, b_a, b_b, Wdq, Wiuq, Ww, Wk, Wuq,
           Wod, Wou, kvn_g, kvn_b, qn_g, qn_b, sink_logits):
    x2 = x.reshape(S, D)
    w4 = jnp.concatenate([Wkva, Wkvb, Wza, Wzb], axis=1)
    w4h = w4.astype(BF16)
    w4l = (w4 - w4h.astype(F32)).astype(BF16)
    wdqw = jnp.concatenate([Wdq, Ww, jnp.zeros((D, 124), F32)],
                           axis=1).astype(BF16)
    wuqh = Wuq.astype(BF16)
    wuql = (Wuq - wuqh.astype(F32)).astype(BF16)
    wod2 = Wod.reshape(NG * 512, DG)
    wodh = wod2.astype(BF16)
    wodl = (wod2 - wodh.astype(F32)).astype(BF16)
    wouh = Wou.astype(BF16)
    woul = (Wou - wouh.astype(F32)).astype(BF16)

    o1, o2, o3 = pl.pallas_call(
        _proj_kernel,
        grid=(NB,),
        in_specs=[pl.BlockSpec((RB, D), lambda i: (i, 0)),
                  pl.BlockSpec((D, 4 * HD), lambda i: (0, 0)),
                  pl.BlockSpec((D, 4 * HD), lambda i: (0, 0)),
                  pl.BlockSpec((D, DC + 128), lambda i: (0, 0))],
        out_specs=[pl.BlockSpec((RB, 4 * HD), lambda i: (i, 0)),
                   pl.BlockSpec((RB, DC), lambda i: (i, 0)),
                   pl.BlockSpec((RB, 128), lambda i: (i, 0))],
        out_shape=[jax.ShapeDtypeStruct((S, 4 * HD), F32),
                   jax.ShapeDtypeStruct((S, DC), F32),
                   jax.ShapeDtypeStruct((S, 128), F32)],
    )(x2, w4h, w4l, wdqw)

    kc, kp = pl.pallas_call(
        _compress_kernel,
        in_specs=[pl.BlockSpec((S, 4 * HD), lambda: (0, 0)),
                  pl.BlockSpec((SK, M_ * D), lambda: (0, 0)),
                  pl.BlockSpec((D, CI), lambda: (0, 0)),
                  pl.BlockSpec((M_, HD), lambda: (0, 0)),
                  pl.BlockSpec((M_, HD), lambda: (0, 0)),
                  pl.BlockSpec((1, HD), lambda: (0, 0)),
                  pl.BlockSpec((1, HD), lambda: (0, 0))],
        out_specs=[pl.BlockSpec((SK, HD), lambda: (0, 0)),
                   pl.BlockSpec((SK, CI), lambda: (0, 0))],
        out_shape=[jax.ShapeDtypeStruct((SK, HD), F32),
                   jax.ShapeDtypeStruct((SK, CI), F32)],
    )(o1, x.reshape(SK, M_ * D), Wk, b_a, b_b,
      kvn_g.reshape(1, HD), kvn_b.reshape(1, HD))

    att = pl.pallas_call(
        _attn_kernel,
        grid=(NB,),
        in_specs=[pl.BlockSpec((RB, DC), lambda i: (i, 0)),
                  pl.BlockSpec((RB, 128), lambda i: (i, 0)),
                  pl.BlockSpec((SK, CI), lambda i: (0, 0)),
                  pl.BlockSpec((SK, HD), lambda i: (0, 0)),
                  pl.BlockSpec((DC, NH * HD), lambda i: (0, 0)),
                  pl.BlockSpec((DC, NH * HD), lambda i: (0, 0)),
                  pl.BlockSpec((DC, NHI * CI), lambda i: (0, 0)),
                  pl.BlockSpec((1, HD), lambda i: (0, 0)),
                  pl.BlockSpec((1, HD), lambda i: (0, 0)),
                  pl.BlockSpec((1, NH), lambda i: (0, 0))],
        out_specs=pl.BlockSpec((RB, NH * HD), lambda i: (i, 0)),
        out_shape=jax.ShapeDtypeStruct((S, NH * HD), F32),
    )(o2, o3, kp, kc, wuqh, wuql, Wiuq.astype(BF16),
      qn_g.reshape(1, HD), qn_b.reshape(1, HD), sink_logits.reshape(1, NH))

    fin = pl.pallas_call(
        _out_kernel,
        grid=(NB,),
        in_specs=[pl.BlockSpec((RB, NH * HD), lambda i: (i, 0)),
                  pl.BlockSpec((NG * 512, DG), lambda i: (0, 0)),
                  pl.BlockSpec((NG * 512, DG), lambda i: (0, 0)),
                  pl.BlockSpec((NG * DG, D), lambda i: (0, 0)),
                  pl.BlockSpec((NG * DG, D), lambda i: (0, 0))],
        out_specs=pl.BlockSpec((RB, D), lambda i: (i, 0)),
        out_shape=jax.ShapeDtypeStruct((S, D), F32),
    )(att, wodh, wodl, wouh, woul)

    return fin.reshape(1, S, D)


# speedup vs baseline: 1.0819x; 1.0819x over previous
import math

import jax
import jax.numpy as jnp
from jax import lax
from jax.experimental import pallas as pl
from jax.experimental.pallas import tpu as pltpu

F32 = jnp.float32
S = 2048
D = 2048
HD = 128
NH = 16
DC = 512
M_ = 4
TOPK = 256
CI = 64
NHI = 4
NG = 4
DG = 1024
RD = 64
EPS = 1e-6
SK = S // M_
RB = 256
NB = S // RB
NEG = -1e30
WBIG = 4 * HD + DC + CI + NHI + 60


PREC = lax.Precision.HIGHEST
BF16 = jnp.bfloat16


def _dot(a, b):
    return lax.dot_general(a, b, (((1,), (0,)), ((), ())),
                           precision=PREC, preferred_element_type=F32)


def _dot_t(a, b):
    return lax.dot_general(a, b, (((1,), (1,)), ((), ())),
                           precision=PREC, preferred_element_type=F32)


_DIMS = (((1,), (0,)), ((), ()))
_DIMS_T = (((1,), (1,)), ((), ()))


def _split_bf(a):
    hi = a.astype(BF16)
    lo = (a - hi.astype(F32)).astype(BF16)
    return hi, lo


def _dot1(a, b, dims=_DIMS):
    return lax.dot_general(a, b, dims, preferred_element_type=F32)


def _dot_ps(a, bh, bl, dims=_DIMS):
    ah, al = _split_bf(a)
    return (_dot1(ah, bh, dims) + _dot1(ah, bl, dims)
            + _dot1(al, bh, dims))


def _dot_h(a, b):
    bh, bl = _split_bf(b)
    return _dot_ps(a, bh, bl)


def _dot_rs(a_bf, b):
    bh, bl = _split_bf(b)
    return _dot1(a_bf, bh) + _dot1(a_bf, bl)


def _dot_bf(a, b):
    return lax.dot_general(a.astype(BF16), b.astype(BF16),
                           (((1,), (0,)), ((), ())),
                           preferred_element_type=F32)


def _dot_t_bf(a, b):
    return lax.dot_general(a.astype(BF16), b.astype(BF16),
                           (((1,), (1,)), ((), ())),
                           preferred_element_type=F32)


def _proj_kernel(x_ref, w4_ref, wdq_ref, o1_ref, o2_ref, o3_ref):
    xh, xl = _split_bf(x_ref[...])
    w4h, w4l = _split_bf(w4_ref[...])
    o1_ref[...] = (_dot1(xh, w4h) + _dot1(xh, w4l) + _dot1(xl, w4h))
    p2 = _dot1(xh, wdq_ref[...])
    o2_ref[...] = p2[:, 0:DC]
    o3_ref[...] = p2[:, DC:DC + 128]


def _compress_kernel(cz_ref, xr_ref, wk_ref, ba_ref, bb_ref, g_ref, b_ref,
                     kc_ref, kp_ref):
    cz = cz_ref[...]
    c_a = cz[:, 0:HD]
    c_b = cz[:, HD:2 * HD]
    z_a = cz[:, 2 * HD:3 * HD]
    z_b = cz[:, 3 * HD:4 * HD]
    r4 = lax.broadcasted_iota(jnp.int32, (S, M_), 0)
    j4 = lax.broadcasted_iota(jnp.int32, (S, M_), 1)
    tsel = ((r4 % M_) == j4).astype(BF16)
    ba = _dot_rs(tsel, ba_ref[...])
    bb = _dot_rs(tsel, bb_ref[...])
    ea = jnp.exp(z_a + ba)
    eb = jnp.exp(z_b + bb)
    nn = lax.broadcasted_iota(jnp.int32, (SK, S), 0)
    rr = lax.broadcasted_iota(jnp.int32, (SK, S), 1)
    dm = ((rr // M_) == nn).astype(BF16)
    ds = ((rr // M_) == (nn - 1)).astype(BF16)
    num = _dot_rs(dm, ea * c_a) + _dot_rs(ds, eb * c_b)
    den = _dot_rs(dm, ea) + _dot_rs(ds, eb)
    comp = num / den
    mu = jnp.mean(comp, axis=1, keepdims=True)
    var = jnp.mean((comp - mu) ** 2, axis=1, keepdims=True)
    kc_ref[...] = ((comp - mu) * lax.rsqrt(var + EPS) * g_ref[...]
                   + b_ref[...])
    xr = xr_ref[...]
    ksum = xr[:, 0:D]
    for j in range(1, M_):
        ksum = ksum + xr[:, j * D:(j + 1) * D]
    korig = ksum * (1.0 / M_)
    kp_ref[...] = _dot_bf(korig, wk_ref[...])


def _attn_kernel(cq_ref, misc_ref, kp_ref, kc_ref, wuq_ref,
                 wiuq_ref, qng_ref, qnb_ref, sink_ref, out_ref):
    pid = pl.program_id(0)
    cq = cq_ref[...]
    qi = _dot1(cq.astype(BF16), wiuq_ref[...])
    kp = kp_ref[...].astype(BF16)
    score = jnp.zeros((RB, SK), F32)
    for h in range(NHI):
        dh = _dot1(qi[:, h * CI:(h + 1) * CI].astype(BF16), kp, _DIMS_T)
        wh = misc_ref[:, h:h + 1]
        score = score + jnp.maximum(dh, 0.0) * wh
    rows = lax.broadcasted_iota(jnp.int32, (RB, SK), 0) + pid * RB
    cols = lax.broadcasted_iota(jnp.int32, (RB, SK), 1)
    score = jnp.where(cols < rows, score, NEG)

    bits = lax.bitcast_convert_type(score, jnp.int32)
    int_min = jnp.int32(-2**31)
    code = jnp.where(bits >= 0, bits + int_min, -bits)

    def body(i, carry):
        sel, cand, krem = carry
        b = 31 - i
        bit = (lax.shift_right_logical(code, b) & 1).astype(F32)
        ones = cand * bit
        c1 = jnp.sum(ones, axis=1, keepdims=True)
        ta = (c1 < krem).astype(F32)
        sel = sel + ones * ta
        krem = krem - c1 * ta
        cand = ones + ta * (cand - 2.0 * ones)
        return sel, cand, krem

    sel0 = jnp.zeros((RB, SK), F32)
    cand0 = jnp.ones((RB, SK), F32)
    krem0 = jnp.full((RB, 1), float(TOPK), F32)
    sel, cand, krem = lax.fori_loop(0, 32, body, (sel0, cand0, krem0))
    li = lax.broadcasted_iota(jnp.int32, (SK, SK), 0)
    lj = lax.broadcasted_iota(jnp.int32, (SK, SK), 1)
    lt = (li < lj).astype(BF16)
    eqr = _dot1(cand.astype(BF16), lt)
    sel = sel + cand * (eqr < krem).astype(F32)

    keep = (sel > 0.5) & (rows < cols * M_)
    abias = jnp.where(keep, 0.0, NEG)

    cqh, cql = _split_bf(cq)
    wuqh, wuql = _split_bf(wuq_ref[...])
    qall = (_dot1(cqh, wuqh) + _dot1(cqh, wuql)
            + _dot1(cql, wuqh))
    kc = kc_ref[...]
    kch, kcl = _split_bf(kc)
    lidx = lax.broadcasted_iota(jnp.int32, (RB, HD), 1)
    srow = (lax.broadcasted_iota(jnp.int32, (RB, HD), 0) + pid * RB
            ).astype(F32)
    expo = ((lidx - RD) & (~1)).astype(F32) * (1.0 / RD)
    theta = jnp.exp(expo * (-math.log(10000.0)))
    ang = srow * theta
    cosf = jnp.where(lidx < RD, 1.0, jnp.cos(ang))
    sinf = jnp.where(lidx < RD, 0.0,
                     jnp.where((lidx & 1) == 0, -jnp.sin(ang), jnp.sin(ang)))
    even = (lidx & 1) == 0
    inv = 1.0 / math.sqrt(float(HD))
    sink = jnp.exp(sink_ref[...])
    for h in range(NH):
        qh = qall[:, h * HD:(h + 1) * HD]
        mu = jnp.mean(qh, axis=1, keepdims=True)
        var = jnp.mean((qh - mu) ** 2, axis=1, keepdims=True)
        qh = (qh - mu) * lax.rsqrt(var + EPS) * qng_ref[...] + qnb_ref[...]
        qsw = jnp.where(even, pltpu.roll(qh, HD - 1, 1),
                        pltpu.roll(qh, 1, 1))
        qh = qh * cosf + qsw * sinf
        qhh, qhl = _split_bf(qh)
        sc_h = (_dot1(qhh, kch, _DIMS_T) + _dot1(qhh, kcl, _DIMS_T)
                + _dot1(qhl, kch, _DIMS_T)) * inv + abias
        es = jnp.exp(sc_h)
        den = jnp.sum(es, axis=1, keepdims=True) + sink[:, h:h + 1]
        p = es / den
        out_ref[:, h * HD:(h + 1) * HD] = _dot_ps(p, kch, kcl)


def _out_kernel(o_ref, wod_ref, wou_ref, f_ref):
    o = o_ref[...]
    acc = jnp.zeros((RB, D), F32)
    gi = (NH * HD) // NG
    for g in range(NG):
        dg = _dot_h(o[:, g * gi:(g + 1) * gi],
                    wod_ref[g * gi:(g + 1) * gi, :])
        acc = acc + _dot_h(dg, wou_ref[g * DG:(g + 1) * DG, :])
    f_ref[...] = acc


def kernel(x, Wkva, Wkvb, Wza, Wzb, b_a, b_b, Wdq, Wiuq, Ww, Wk, Wuq,
           Wod, Wou, kvn_g, kvn_b, qn_g, qn_b, sink_logits):
    x2 = x.reshape(S, D)
    w4 = jnp.concatenate([Wkva, Wkvb, Wza, Wzb], axis=1)
    wdqw = jnp.concatenate([Wdq, Ww, jnp.zeros((D, 124), F32)],
                           axis=1).astype(BF16)

    o1, o2, o3 = pl.pallas_call(
        _proj_kernel,
        grid=(NB,),
        in_specs=[pl.BlockSpec((RB, D), lambda i: (i, 0)),
                  pl.BlockSpec((D, 4 * HD), lambda i: (0, 0)),
                  pl.BlockSpec((D, DC + 128), lambda i: (0, 0))],
        out_specs=[pl.BlockSpec((RB, 4 * HD), lambda i: (i, 0)),
                   pl.BlockSpec((RB, DC), lambda i: (i, 0)),
                   pl.BlockSpec((RB, 128), lambda i: (i, 0))],
        out_shape=[jax.ShapeDtypeStruct((S, 4 * HD), F32),
                   jax.ShapeDtypeStruct((S, DC), F32),
                   jax.ShapeDtypeStruct((S, 128), F32)],
    )(x2, w4, wdqw)

    kc, kp = pl.pallas_call(
        _compress_kernel,
        in_specs=[pl.BlockSpec((S, 4 * HD), lambda: (0, 0)),
                  pl.BlockSpec((SK, M_ * D), lambda: (0, 0)),
                  pl.BlockSpec((D, CI), lambda: (0, 0)),
                  pl.BlockSpec((M_, HD), lambda: (0, 0)),
                  pl.BlockSpec((M_, HD), lambda: (0, 0)),
                  pl.BlockSpec((1, HD), lambda: (0, 0)),
                  pl.BlockSpec((1, HD), lambda: (0, 0))],
        out_specs=[pl.BlockSpec((SK, HD), lambda: (0, 0)),
                   pl.BlockSpec((SK, CI), lambda: (0, 0))],
        out_shape=[jax.ShapeDtypeStruct((SK, HD), F32),
                   jax.ShapeDtypeStruct((SK, CI), F32)],
    )(o1, x.reshape(SK, M_ * D), Wk, b_a, b_b,
      kvn_g.reshape(1, HD), kvn_b.reshape(1, HD))

    att = pl.pallas_call(
        _attn_kernel,
        grid=(NB,),
        in_specs=[pl.BlockSpec((RB, DC), lambda i: (i, 0)),
                  pl.BlockSpec((RB, 128), lambda i: (i, 0)),
                  pl.BlockSpec((SK, CI), lambda i: (0, 0)),
                  pl.BlockSpec((SK, HD), lambda i: (0, 0)),
                  pl.BlockSpec((DC, NH * HD), lambda i: (0, 0)),
                  pl.BlockSpec((DC, NHI * CI), lambda i: (0, 0)),
                  pl.BlockSpec((1, HD), lambda i: (0, 0)),
                  pl.BlockSpec((1, HD), lambda i: (0, 0)),
                  pl.BlockSpec((1, NH), lambda i: (0, 0))],
        out_specs=pl.BlockSpec((RB, NH * HD), lambda i: (i, 0)),
        out_shape=jax.ShapeDtypeStruct((S, NH * HD), F32),
    )(o2, o3, kp, kc, Wuq, Wiuq.astype(BF16),
      qn_g.reshape(1, HD), qn_b.reshape(1, HD), sink_logits.reshape(1, NH))

    fin = pl.pallas_call(
        _out_kernel,
        grid=(NB,),
        in_specs=[pl.BlockSpec((RB, NH * HD), lambda i: (i, 0)),
                  pl.BlockSpec((NG * 512, DG), lambda i: (0, 0)),
                  pl.BlockSpec((NG * DG, D), lambda i: (0, 0))],
        out_specs=pl.BlockSpec((RB, D), lambda i: (i, 0)),
        out_shape=jax.ShapeDtypeStruct((S, D), F32),
    )(att, Wod.reshape(NG * 512, DG), Wou)

    return fin.reshape(1, S, D)


# speedup vs baseline: 1.3427x; 1.2411x over previous
import math

import jax
import jax.numpy as jnp
from jax import lax
from jax.experimental import pallas as pl
from jax.experimental.pallas import tpu as pltpu

F32 = jnp.float32
S = 2048
D = 2048
HD = 128
NH = 16
DC = 512
M_ = 4
TOPK = 256
CI = 64
NHI = 4
NG = 4
DG = 1024
RD = 64
EPS = 1e-6
SK = S // M_
RB = 256
NB = S // RB
NEG = -1e30
WBIG = 4 * HD + DC + CI + NHI + 60


PREC = lax.Precision.HIGHEST
BF16 = jnp.bfloat16


def _dot(a, b):
    return lax.dot_general(a, b, (((1,), (0,)), ((), ())),
                           precision=PREC, preferred_element_type=F32)


def _dot_t(a, b):
    return lax.dot_general(a, b, (((1,), (1,)), ((), ())),
                           precision=PREC, preferred_element_type=F32)


_DIMS = (((1,), (0,)), ((), ()))
_DIMS_T = (((1,), (1,)), ((), ()))


def _split_bf(a):
    hi = a.astype(BF16)
    lo = (a - hi.astype(F32)).astype(BF16)
    return hi, lo


def _dot1(a, b, dims=_DIMS):
    return lax.dot_general(a, b, dims, preferred_element_type=F32)


def _dot_ps(a, bh, bl, dims=_DIMS):
    ah, al = _split_bf(a)
    return (_dot1(ah, bh, dims) + _dot1(ah, bl, dims)
            + _dot1(al, bh, dims))


def _dot_h(a, b):
    bh, bl = _split_bf(b)
    return _dot_ps(a, bh, bl)


def _dot_rs(a_bf, b):
    bh, bl = _split_bf(b)
    return _dot1(a_bf, bh) + _dot1(a_bf, bl)


def _dot_bf(a, b):
    return lax.dot_general(a.astype(BF16), b.astype(BF16),
                           (((1,), (0,)), ((), ())),
                           preferred_element_type=F32)


def _dot_t_bf(a, b):
    return lax.dot_general(a.astype(BF16), b.astype(BF16),
                           (((1,), (1,)), ((), ())),
                           preferred_element_type=F32)


def _proj_kernel(x_ref, w4_ref, wdq_ref, o1_ref, o2_ref, o3_ref):
    xh, xl = _split_bf(x_ref[...])
    w4h, w4l = _split_bf(w4_ref[...])
    o1_ref[...] = (_dot1(xh, w4h) + _dot1(xh, w4l) + _dot1(xl, w4h))
    p2 = _dot1(xh, wdq_ref[...])
    o2_ref[...] = p2[:, 0:DC]
    o3_ref[...] = p2[:, DC:DC + 128]


def _compress_kernel(cz_ref, xr_ref, wk_ref, ba_ref, bb_ref, g_ref, b_ref,
                     kc_ref, kp_ref):
    cz = cz_ref[...]
    c_a = cz[:, 0:HD]
    c_b = cz[:, HD:2 * HD]
    z_a = cz[:, 2 * HD:3 * HD]
    z_b = cz[:, 3 * HD:4 * HD]
    r4 = lax.broadcasted_iota(jnp.int32, (S, M_), 0)
    j4 = lax.broadcasted_iota(jnp.int32, (S, M_), 1)
    tsel = ((r4 % M_) == j4).astype(BF16)
    ba = _dot_rs(tsel, ba_ref[...])
    bb = _dot_rs(tsel, bb_ref[...])
    ea = jnp.exp(z_a + ba)
    eb = jnp.exp(z_b + bb)
    nn = lax.broadcasted_iota(jnp.int32, (SK, S), 0)
    rr = lax.broadcasted_iota(jnp.int32, (SK, S), 1)
    dm = ((rr // M_) == nn).astype(BF16)
    ds = ((rr // M_) == (nn - 1)).astype(BF16)
    num = _dot_rs(dm, ea * c_a) + _dot_rs(ds, eb * c_b)
    den = _dot_rs(dm, ea) + _dot_rs(ds, eb)
    comp = num / den
    mu = jnp.mean(comp, axis=1, keepdims=True)
    var = jnp.mean((comp - mu) ** 2, axis=1, keepdims=True)
    kc_ref[...] = ((comp - mu) * lax.rsqrt(var + EPS) * g_ref[...]
                   + b_ref[...])
    xr = xr_ref[...]
    ksum = xr[:, 0:D]
    for j in range(1, M_):
        ksum = ksum + xr[:, j * D:(j + 1) * D]
    korig = ksum * (1.0 / M_)
    kp_ref[...] = _dot_bf(korig, wk_ref[...])


def _attn_kernel(cq_ref, misc_ref, kp_ref, kc_ref, wuq_ref,
                 wiuq_ref, qng_ref, qnb_ref, sink_ref, out_ref):
    pid = pl.program_id(0)
    cq = cq_ref[...]
    qi = _dot1(cq.astype(BF16), wiuq_ref[...])
    kp = kp_ref[...].astype(BF16)
    score = jnp.zeros((RB, SK), F32)
    for h in range(NHI):
        dh = _dot1(qi[:, h * CI:(h + 1) * CI].astype(BF16), kp, _DIMS_T)
        wh = misc_ref[:, h:h + 1]
        score = score + jnp.maximum(dh, 0.0) * wh
    rows = lax.broadcasted_iota(jnp.int32, (RB, SK), 0) + pid * RB
    cols = lax.broadcasted_iota(jnp.int32, (RB, SK), 1)
    score = jnp.where(cols < rows, score, NEG)

    bits = lax.bitcast_convert_type(score, jnp.int32)
    int_min = jnp.int32(-2**31)
    code = jnp.where(bits >= 0, bits + int_min, -bits)

    def body(i, carry):
        sel, cand, krem = carry
        b = 31 - i
        bit = (lax.shift_right_logical(code, b) & 1).astype(F32)
        ones = cand * bit
        c1 = jnp.sum(ones, axis=1, keepdims=True)
        ta = (c1 < krem).astype(F32)
        sel = sel + ones * ta
        krem = krem - c1 * ta
        cand = ones + ta * (cand - 2.0 * ones)
        return sel, cand, krem

    sel0 = jnp.zeros((RB, SK), F32)
    cand0 = jnp.ones((RB, SK), F32)
    krem0 = jnp.full((RB, 1), float(TOPK), F32)
    sel, cand, krem = lax.fori_loop(0, 32, body, (sel0, cand0, krem0))
    li = lax.broadcasted_iota(jnp.int32, (SK, SK), 0)
    lj = lax.broadcasted_iota(jnp.int32, (SK, SK), 1)
    lt = (li < lj).astype(BF16)
    eqr = _dot1(cand.astype(BF16), lt)
    sel = sel + cand * (eqr < krem).astype(F32)

    keep = (sel > 0.5) & (rows < cols * M_)
    abias = jnp.where(keep, 0.0, NEG)

    cqh, cql = _split_bf(cq)
    wuqh, wuql = _split_bf(wuq_ref[...])
    qall = (_dot1(cqh, wuqh) + _dot1(cqh, wuql)
            + _dot1(cql, wuqh))
    kc = kc_ref[...]
    kch, kcl = _split_bf(kc)
    lidx = lax.broadcasted_iota(jnp.int32, (RB, HD), 1)
    srow = (lax.broadcasted_iota(jnp.int32, (RB, HD), 0) + pid * RB
            ).astype(F32)
    expo = ((lidx - RD) & (~1)).astype(F32) * (1.0 / RD)
    theta = jnp.exp(expo * (-math.log(10000.0)))
    ang = srow * theta
    cosf = jnp.where(lidx < RD, 1.0, jnp.cos(ang))
    sinf = jnp.where(lidx < RD, 0.0,
                     jnp.where((lidx & 1) == 0, -jnp.sin(ang), jnp.sin(ang)))
    even = (lidx & 1) == 0
    inv = 1.0 / math.sqrt(float(HD))
    sink = jnp.exp(sink_ref[...])
    for h in range(NH):
        qh = qall[:, h * HD:(h + 1) * HD]
        mu = jnp.mean(qh, axis=1, keepdims=True)
        var = jnp.mean((qh - mu) ** 2, axis=1, keepdims=True)
        qh = (qh - mu) * lax.rsqrt(var + EPS) * qng_ref[...] + qnb_ref[...]
        qsw = jnp.where(even, pltpu.roll(qh, HD - 1, 1),
                        pltpu.roll(qh, 1, 1))
        qh = qh * cosf + qsw * sinf
        qhh, qhl = _split_bf(qh)
        sc_h = (_dot1(qhh, kch, _DIMS_T) + _dot1(qhh, kcl, _DIMS_T)
                + _dot1(qhl, kch, _DIMS_T)) * inv + abias
        es = jnp.exp(sc_h)
        den = jnp.sum(es, axis=1, keepdims=True) + sink[:, h:h + 1]
        p = es / den
        out_ref[:, h * HD:(h + 1) * HD] = _dot_ps(p, kch, kcl)


def _out_kernel(o_ref, wod_ref, wou_ref, f_ref):
    o = o_ref[...]
    acc = jnp.zeros((RB, D), F32)
    gi = (NH * HD) // NG
    for g in range(NG):
        dg = _dot1(o[:, g * gi:(g + 1) * gi].astype(BF16),
                   wod_ref[g * gi:(g + 1) * gi, :].astype(BF16))
        acc = acc + _dot1(dg.astype(BF16),
                          wou_ref[g * DG:(g + 1) * DG, :].astype(BF16))
    f_ref[...] = acc


def kernel(x, Wkva, Wkvb, Wza, Wzb, b_a, b_b, Wdq, Wiuq, Ww, Wk, Wuq,
           Wod, Wou, kvn_g, kvn_b, qn_g, qn_b, sink_logits):
    x2 = x.reshape(S, D)
    w4 = jnp.concatenate([Wkva, Wkvb, Wza, Wzb], axis=1)
    wdqw = jnp.concatenate([Wdq, Ww, jnp.zeros((D, 124), F32)],
                           axis=1).astype(BF16)

    o1, o2, o3 = pl.pallas_call(
        _proj_kernel,
        grid=(NB,),
        in_specs=[pl.BlockSpec((RB, D), lambda i: (i, 0)),
                  pl.BlockSpec((D, 4 * HD), lambda i: (0, 0)),
                  pl.BlockSpec((D, DC + 128), lambda i: (0, 0))],
        out_specs=[pl.BlockSpec((RB, 4 * HD), lambda i: (i, 0)),
                   pl.BlockSpec((RB, DC), lambda i: (i, 0)),
                   pl.BlockSpec((RB, 128), lambda i: (i, 0))],
        out_shape=[jax.ShapeDtypeStruct((S, 4 * HD), F32),
                   jax.ShapeDtypeStruct((S, DC), F32),
                   jax.ShapeDtypeStruct((S, 128), F32)],
    )(x2, w4, wdqw)

    kc, kp = pl.pallas_call(
        _compress_kernel,
        in_specs=[pl.BlockSpec((S, 4 * HD), lambda: (0, 0)),
                  pl.BlockSpec((SK, M_ * D), lambda: (0, 0)),
                  pl.BlockSpec((D, CI), lambda: (0, 0)),
                  pl.BlockSpec((M_, HD), lambda: (0, 0)),
                  pl.BlockSpec((M_, HD), lambda: (0, 0)),
                  pl.BlockSpec((1, HD), lambda: (0, 0)),
                  pl.BlockSpec((1, HD), lambda: (0, 0))],
        out_specs=[pl.BlockSpec((SK, HD), lambda: (0, 0)),
                   pl.BlockSpec((SK, CI), lambda: (0, 0))],
        out_shape=[jax.ShapeDtypeStruct((SK, HD), F32),
                   jax.ShapeDtypeStruct((SK, CI), F32)],
    )(o1, x.reshape(SK, M_ * D), Wk, b_a, b_b,
      kvn_g.reshape(1, HD), kvn_b.reshape(1, HD))

    att = pl.pallas_call(
        _attn_kernel,
        grid=(NB,),
        in_specs=[pl.BlockSpec((RB, DC), lambda i: (i, 0)),
                  pl.BlockSpec((RB, 128), lambda i: (i, 0)),
                  pl.BlockSpec((SK, CI), lambda i: (0, 0)),
                  pl.BlockSpec((SK, HD), lambda i: (0, 0)),
                  pl.BlockSpec((DC, NH * HD), lambda i: (0, 0)),
                  pl.BlockSpec((DC, NHI * CI), lambda i: (0, 0)),
                  pl.BlockSpec((1, HD), lambda i: (0, 0)),
                  pl.BlockSpec((1, HD), lambda i: (0, 0)),
                  pl.BlockSpec((1, NH), lambda i: (0, 0))],
        out_specs=pl.BlockSpec((RB, NH * HD), lambda i: (i, 0)),
        out_shape=jax.ShapeDtypeStruct((S, NH * HD), F32),
    )(o2, o3, kp, kc, Wuq, Wiuq.astype(BF16),
      qn_g.reshape(1, HD), qn_b.reshape(1, HD), sink_logits.reshape(1, NH))

    fin = pl.pallas_call(
        _out_kernel,
        grid=(NB,),
        in_specs=[pl.BlockSpec((RB, NH * HD), lambda i: (i, 0)),
                  pl.BlockSpec((NG * 512, DG), lambda i: (0, 0)),
                  pl.BlockSpec((NG * DG, D), lambda i: (0, 0))],
        out_specs=pl.BlockSpec((RB, D), lambda i: (i, 0)),
        out_shape=jax.ShapeDtypeStruct((S, D), F32),
    )(att, Wod.reshape(NG * 512, DG), Wou)

    return fin.reshape(1, S, D)
